# baseline (speedup 1.0000x reference)
import jax
import jax.numpy as jnp
from jax.experimental import pallas as pl
from jax.experimental.pallas import tpu as pltpu

_BOUNDS = (0, 2048, 8192)


def _dma_copy_kernel(in_ref, out_ref, vbuf, in_sem, out_sem):
    n = len(_BOUNDS) - 1
    loads = [
        pltpu.make_async_copy(
            in_ref.at[pl.ds(_BOUNDS[k], _BOUNDS[k + 1] - _BOUNDS[k])],
            vbuf.at[pl.ds(_BOUNDS[k], _BOUNDS[k + 1] - _BOUNDS[k])],
            in_sem.at[k],
        )
        for k in range(n)
    ]
    stores = [
        pltpu.make_async_copy(
            vbuf.at[pl.ds(_BOUNDS[k], _BOUNDS[k + 1] - _BOUNDS[k])],
            out_ref.at[pl.ds(_BOUNDS[k], _BOUNDS[k + 1] - _BOUNDS[k])],
            out_sem.at[k],
        )
        for k in range(n)
    ]
    for k in range(n):
        loads[k].start()
    for k in range(n):
        loads[k].wait()
        stores[k].start()
    for k in range(n):
        stores[k].wait()


def kernel(seq_len, position_embedding):
    del seq_len
    M, H = position_embedding.shape
    n = len(_BOUNDS) - 1
    return pl.pallas_call(
        _dma_copy_kernel,
        in_specs=[pl.BlockSpec(memory_space=pltpu.MemorySpace.HBM)],
        out_specs=pl.BlockSpec(memory_space=pltpu.MemorySpace.HBM),
        out_shape=jax.ShapeDtypeStruct((M, H), position_embedding.dtype),
        scratch_shapes=[
            pltpu.VMEM((M, H), position_embedding.dtype),
            pltpu.SemaphoreType.DMA((n,)),
            pltpu.SemaphoreType.DMA((n,)),
        ],
    )(position_embedding)


# speedup vs baseline: 1.0606x; 1.0606x over previous
import jax
import jax.numpy as jnp
from jax.experimental import pallas as pl
from jax.experimental.pallas import tpu as pltpu

_BOUNDS = (0, 4096, 8192)


def _dma_copy_kernel(in_ref, out_ref, vbuf, in_sem, out_sem):
    n = len(_BOUNDS) - 1
    loads = [
        pltpu.make_async_copy(
            in_ref.at[pl.ds(_BOUNDS[k], _BOUNDS[k + 1] - _BOUNDS[k])],
            vbuf.at[pl.ds(_BOUNDS[k], _BOUNDS[k + 1] - _BOUNDS[k])],
            in_sem.at[k],
        )
        for k in range(n)
    ]
    stores = [
        pltpu.make_async_copy(
            vbuf.at[pl.ds(_BOUNDS[k], _BOUNDS[k + 1] - _BOUNDS[k])],
            out_ref.at[pl.ds(_BOUNDS[k], _BOUNDS[k + 1] - _BOUNDS[k])],
            out_sem.at[k],
        )
        for k in range(n)
    ]
    for k in range(n):
        loads[k].start()
    for k in range(n):
        loads[k].wait()
        stores[k].start()
    for k in range(n):
        stores[k].wait()


def kernel(seq_len, position_embedding):
    del seq_len
    M, H = position_embedding.shape
    n = len(_BOUNDS) - 1
    return pl.pallas_call(
        _dma_copy_kernel,
        in_specs=[pl.BlockSpec(memory_space=pltpu.MemorySpace.HBM)],
        out_specs=pl.BlockSpec(memory_space=pltpu.MemorySpace.HBM),
        out_shape=jax.ShapeDtypeStruct((M, H), position_embedding.dtype),
        scratch_shapes=[
            pltpu.VMEM((M, H), position_embedding.dtype),
            pltpu.SemaphoreType.DMA((n,)),
            pltpu.SemaphoreType.DMA((n,)),
        ],
    )(position_embedding)
